# baseline (speedup 1.0000x reference)
import functools

import jax
import jax.numpy as jnp
from jax import lax
from jax.experimental import pallas as pl
from jax.experimental.pallas import tpu as pltpu
from jax.experimental.pallas import tpu_sc as plsc

N = 10000
E = 320000
D_IN = 128
H = 8
D = 16
HID = H * D
OUT = 64
WROW = HID + 16
ERW = 16

NP = 10240
NC = 2
NS = 16
NW = NC * NS
EPW = E // NW
C = 80
NCHUNK = EPW // C


def _bcast_lane(v, h):
    idx = jnp.full((16, 1), h, dtype=jnp.int32)
    dn = lax.GatherDimensionNumbers(
        offset_dims=(), collapsed_slice_dims=(0,), start_index_map=(0,))
    return lax.gather(v, idx, dn, (1,),
                      mode=lax.GatherScatterMode.PROMISE_IN_BOUNDS)


RB = 2000
NG = N // RB


def _prep_body(x_ref, w_ref, al_ref, ar_ref, big_ref, er_ref):
    feat = jnp.dot(x_ref[...], w_ref[...], preferred_element_type=jnp.float32)
    f3 = feat.reshape(RB, H, D)
    el = jnp.sum(f3 * al_ref[...][None], axis=-1)
    er = jnp.sum(f3 * ar_ref[...][None], axis=-1)
    z8 = jnp.zeros((RB, 8), dtype=jnp.float32)
    big_ref[...] = jnp.concatenate([feat, el, z8], axis=1)
    er_ref[...] = jnp.concatenate([er, z8], axis=1)


def _prep(x, w, al, ar):
    return pl.pallas_call(
        _prep_body,
        grid=(NG,),
        in_specs=[
            pl.BlockSpec((RB, D_IN), lambda i: (i, 0)),
            pl.BlockSpec((D_IN, HID), lambda i: (0, 0)),
            pl.BlockSpec((H, D), lambda i: (0, 0)),
            pl.BlockSpec((H, D), lambda i: (0, 0)),
        ],
        out_specs=[
            pl.BlockSpec((RB, WROW), lambda i: (i, 0)),
            pl.BlockSpec((RB, ERW), lambda i: (i, 0)),
        ],
        out_shape=[
            jax.ShapeDtypeStruct((N, WROW), jnp.float32),
            jax.ShapeDtypeStruct((N, ERW), jnp.float32),
        ],
    )(x, w, al, ar)


def _edge_body(big_hbm, er_hbm, src_hbm, dst_hbm, zero_hbm, out_hbm,
               src_v, dst_v, rows_v, er_v, orow_v, acc_sh, sem1, sem2):
    c = lax.axis_index("c")
    s = lax.axis_index("s")
    wid = s * NC + c
    rpt = NP // NS

    pltpu.sync_copy(zero_hbm.at[pl.ds(s * rpt, rpt)],
                    acc_sh.at[pl.ds(s * rpt, rpt)])
    plsc.subcore_barrier()

    def chunk(i, carry):
        base = wid * EPW + i * C
        pltpu.sync_copy(src_hbm.at[pl.ds(base, C)], src_v)
        pltpu.sync_copy(dst_hbm.at[pl.ds(base, C)], dst_v)
        g1 = pltpu.async_copy(big_hbm.at[src_v], rows_v, sem1)
        g2 = pltpu.async_copy(er_hbm.at[dst_v], er_v, sem2)
        g1.wait()
        g2.wait()

        @plsc.parallel_loop(0, C, 1, unroll=4)
        def _edge(e):
            el16 = rows_v[e, pl.ds(HID, 16)]
            er16 = er_v[e, :]
            x = el16 + er16
            ev = jnp.where(x >= 0, x, 0.2 * x)
            ee = jnp.exp(ev)
            orow_v[e, pl.ds(HID, 16)] = ee
            for h in range(H):
                bv = _bcast_lane(ee, h)
                orow_v[e, pl.ds(h * D, D)] = rows_v[e, pl.ds(h * D, D)] * bv
        pltpu.sync_copy(orow_v, acc_sh.at[dst_v], add=True)
        return carry

    lax.fori_loop(0, NCHUNK, chunk, 0)
    plsc.subcore_barrier()
    pltpu.sync_copy(acc_sh.at[pl.ds(s * rpt, rpt)],
                    out_hbm.at[pl.ds(c * NP + s * rpt, rpt)])


_edge_pass = functools.partial(
    pl.kernel,
    out_type=jax.ShapeDtypeStruct((NC * NP, WROW), jnp.float32),
    mesh=plsc.VectorSubcoreMesh(core_axis_name="c", subcore_axis_name="s"),
    compiler_params=pltpu.CompilerParams(use_tc_tiling_on_sc=False),
    scratch_types=[
        pltpu.VMEM((C,), jnp.int32),
        pltpu.VMEM((C,), jnp.int32),
        pltpu.VMEM((C, WROW), jnp.float32),
        pltpu.VMEM((C, ERW), jnp.float32),
        pltpu.VMEM((C, WROW), jnp.float32),
        pltpu.VMEM_SHARED((NP, WROW), jnp.float32),
        pltpu.SemaphoreType.DMA,
        pltpu.SemaphoreType.DMA,
    ],
)(_edge_body)


def _gat_out_block(a0_ref, a1_ref, b_ref):
    acc = a0_ref[...] + a1_ref[...]
    ssum = acc[:, HID:HID + H]
    ssum = jnp.where(ssum > 0, ssum, 1.0)
    o = acc[:, :HID].reshape(RB, H, D) / ssum[:, :, None]
    return o.reshape(RB, HID) + b_ref[...]


def _finalize_body(a0_ref, a1_ref, b_ref, g_ref, be_ref, out_ref, st_ref):
    p = pl.program_id(0)
    i = pl.program_id(1)
    o = _gat_out_block(a0_ref, a1_ref, b_ref)

    @pl.when(jnp.logical_and(p == 0, i == 0))
    def _():
        st_ref[...] = jnp.zeros_like(st_ref)

    @pl.when(p == 0)
    def _():
        st_ref[0, :] += jnp.sum(o, axis=0)
        st_ref[1, :] += jnp.sum(o * o, axis=0)

    @pl.when(p == 1)
    def _():
        mu = st_ref[0, :] / N
        var = st_ref[1, :] / N - mu * mu
        on = (o - mu) / jnp.sqrt(var + 1e-3) * g_ref[...] + be_ref[...]
        out_ref[...] = jnp.maximum(on, 0.0)


def _finalize(a0, a1, b, g, be):
    return pl.pallas_call(
        _finalize_body,
        grid=(2, NG),
        in_specs=[
            pl.BlockSpec((RB, WROW), lambda p, i: (i, 0)),
            pl.BlockSpec((RB, WROW), lambda p, i: (i, 0)),
            pl.BlockSpec((HID,), lambda p, i: (0,)),
            pl.BlockSpec((HID,), lambda p, i: (0,)),
            pl.BlockSpec((HID,), lambda p, i: (0,)),
        ],
        out_specs=pl.BlockSpec((RB, HID), lambda p, i: (i, 0)),
        out_shape=jax.ShapeDtypeStruct((N, HID), jnp.float32),
        scratch_shapes=[pltpu.VMEM((8, HID), jnp.float32)],
    )(a0, a1, b, g, be)


def _head_body(a0_ref, a1_ref, b_ref, g_ref, be_ref, h1_ref, lw_ref, lb_ref,
               mask_ref, out_ref, st_ref):
    p = pl.program_id(0)
    i = pl.program_id(1)

    @pl.when(p < 2)
    def _():
        o = _gat_out_block(a0_ref, a1_ref, b_ref)

        @pl.when(jnp.logical_and(p == 0, i == 0))
        def _():
            st_ref[...] = jnp.zeros_like(st_ref)

        @pl.when(p == 0)
        def _():
            st_ref[0, :] += jnp.sum(o, axis=0)
            st_ref[1, :] += jnp.sum(o * o, axis=0)

        @pl.when(p == 1)
        def _():
            mu = st_ref[0, :] / N
            var = st_ref[1, :] / N - mu * mu
            on = (o - mu) / jnp.sqrt(var + 1e-3) * g_ref[...] + be_ref[...]
            h2 = jnp.maximum(on, 0.0) + h1_ref[...]
            st_ref[2, :] += jnp.sum(h2, axis=0)

    @pl.when(p == 2)
    def _():
        pooled = (st_ref[2, :] / N)[None, :]
        m = (mask_ref[...] > 0.5).astype(jnp.float32)
        w = lw_ref[...] * m
        out_ref[...] = (
            jnp.dot(pooled, w.T, preferred_element_type=jnp.float32)
            + lb_ref[...][None, :])


def _head(a0, a1, b, g, be, h1, lw, lb, mask_real):
    return pl.pallas_call(
        _head_body,
        grid=(3, NG),
        in_specs=[
            pl.BlockSpec((RB, WROW), lambda p, i: (i, 0)),
            pl.BlockSpec((RB, WROW), lambda p, i: (i, 0)),
            pl.BlockSpec((HID,), lambda p, i: (0,)),
            pl.BlockSpec((HID,), lambda p, i: (0,)),
            pl.BlockSpec((HID,), lambda p, i: (0,)),
            pl.BlockSpec((RB, HID), lambda p, i: (i, 0)),
            pl.BlockSpec((OUT, HID), lambda p, i: (0, 0)),
            pl.BlockSpec((OUT,), lambda p, i: (0,)),
            pl.BlockSpec((OUT, HID), lambda p, i: (0, 0)),
        ],
        out_specs=pl.BlockSpec((1, OUT), lambda p, i: (0, 0)),
        out_shape=jax.ShapeDtypeStruct((1, OUT), jnp.float32),
        scratch_shapes=[pltpu.VMEM((8, HID), jnp.float32)],
    )(a0, a1, b, g, be, h1, lw, lb, mask_real)


def kernel(h, edge_index, W1, al1, ar1, b1, g1, be1, W2, al2, ar2, b2, g2,
           be2, lw, lb, mask_real):
    src = edge_index[0]
    dst = edge_index[1]
    zeros = jnp.zeros((NP, WROW), dtype=jnp.float32)

    big1, er1 = _prep(h, W1, al1, ar1)
    acc1 = _edge_pass(big1, er1, src, dst, zeros)
    h1 = _finalize(acc1[:N], acc1[NP:NP + N], b1, g1, be1)

    big2, er2 = _prep(h1, W2, al2, ar2)
    acc2 = _edge_pass(big2, er2, src, dst, zeros)
    return _head(acc2[:N], acc2[NP:NP + N], b2, g2, be2, h1, lw, lb,
                 mask_real)


# speedup vs baseline: 1.3184x; 1.3184x over previous
import functools

import jax
import jax.numpy as jnp
from jax import lax
from jax.experimental import pallas as pl
from jax.experimental.pallas import tpu as pltpu
from jax.experimental.pallas import tpu_sc as plsc

N = 10000
E = 320000
D_IN = 128
H = 8
D = 16
HID = H * D
OUT = 64
WROW = HID + 16
ERW = 16

NP = 10240
NC = 2
NS = 16
NW = NC * NS
EPW = E // NW
C = 80
NCHUNK = EPW // C
NPAIR = (NCHUNK - 1) // 2


def _bcast_lane(v, h):
    idx = jnp.full((16, 1), h, dtype=jnp.int32)
    dn = lax.GatherDimensionNumbers(
        offset_dims=(), collapsed_slice_dims=(0,), start_index_map=(0,))
    return lax.gather(v, idx, dn, (1,),
                      mode=lax.GatherScatterMode.PROMISE_IN_BOUNDS)


RB = 2000
NG = N // RB


def _prep_body(x_ref, w_ref, al_ref, ar_ref, big_ref, er_ref):
    feat = jnp.dot(x_ref[...], w_ref[...], preferred_element_type=jnp.float32)
    f3 = feat.reshape(RB, H, D)
    el = jnp.sum(f3 * al_ref[...][None], axis=-1)
    er = jnp.sum(f3 * ar_ref[...][None], axis=-1)
    z8 = jnp.zeros((RB, 8), dtype=jnp.float32)
    big_ref[...] = jnp.concatenate([feat, el, z8], axis=1)
    er_ref[...] = jnp.concatenate([er, z8], axis=1)


def _prep(x, w, al, ar):
    return pl.pallas_call(
        _prep_body,
        grid=(NG,),
        in_specs=[
            pl.BlockSpec((RB, D_IN), lambda i: (i, 0)),
            pl.BlockSpec((D_IN, HID), lambda i: (0, 0)),
            pl.BlockSpec((H, D), lambda i: (0, 0)),
            pl.BlockSpec((H, D), lambda i: (0, 0)),
        ],
        out_specs=[
            pl.BlockSpec((RB, WROW), lambda i: (i, 0)),
            pl.BlockSpec((RB, ERW), lambda i: (i, 0)),
        ],
        out_shape=[
            jax.ShapeDtypeStruct((N, WROW), jnp.float32),
            jax.ShapeDtypeStruct((N, ERW), jnp.float32),
        ],
    )(x, w, al, ar)


def _edge_body(big_hbm, er_hbm, src_hbm, dst_hbm, zero_hbm, out_hbm,
               src_a, dst_a, rows_a, er_a, src_b, dst_b, rows_b, er_b,
               orow_v, acc_sh, sa1, sa2, sb1, sb2):
    c = lax.axis_index("c")
    s = lax.axis_index("s")
    wid = s * NC + c
    rpt = NP // NS

    pltpu.sync_copy(zero_hbm.at[pl.ds(s * rpt, rpt)],
                    acc_sh.at[pl.ds(s * rpt, rpt)])
    plsc.subcore_barrier()

    def fetch(i, src_v, dst_v, rows_v, er_v, s1, s2):
        base = wid * EPW + i * C
        pltpu.sync_copy(src_hbm.at[pl.ds(base, C)], src_v)
        pltpu.sync_copy(dst_hbm.at[pl.ds(base, C)], dst_v)
        pltpu.async_copy(big_hbm.at[src_v], rows_v, s1)
        pltpu.async_copy(er_hbm.at[dst_v], er_v, s2)

    def wait(src_v, dst_v, rows_v, er_v, s1, s2):
        pltpu.make_async_copy(big_hbm.at[src_v], rows_v, s1).wait()
        pltpu.make_async_copy(er_hbm.at[dst_v], er_v, s2).wait()

    def compute(dst_v, rows_v, er_v):
        @plsc.parallel_loop(0, C, 1, unroll=4)
        def _edge(e):
            el16 = rows_v[e, pl.ds(HID, 16)]
            er16 = er_v[e, :]
            x = el16 + er16
            ev = jnp.where(x >= 0, x, 0.2 * x)
            ee = jnp.exp(ev)
            orow_v[e, pl.ds(HID, 16)] = ee
            for h in range(H):
                bv = _bcast_lane(ee, h)
                orow_v[e, pl.ds(h * D, D)] = rows_v[e, pl.ds(h * D, D)] * bv
        pltpu.sync_copy(orow_v, acc_sh.at[dst_v], add=True)

    fetch(0, src_a, dst_a, rows_a, er_a, sa1, sa2)

    def pair(p, carry):
        fetch(2 * p + 1, src_b, dst_b, rows_b, er_b, sb1, sb2)
        wait(src_a, dst_a, rows_a, er_a, sa1, sa2)
        compute(dst_a, rows_a, er_a)
        fetch(2 * p + 2, src_a, dst_a, rows_a, er_a, sa1, sa2)
        wait(src_b, dst_b, rows_b, er_b, sb1, sb2)
        compute(dst_b, rows_b, er_b)
        return carry

    lax.fori_loop(0, NPAIR, pair, 0)
    wait(src_a, dst_a, rows_a, er_a, sa1, sa2)
    compute(dst_a, rows_a, er_a)

    plsc.subcore_barrier()
    pltpu.sync_copy(acc_sh.at[pl.ds(s * rpt, rpt)],
                    out_hbm.at[pl.ds(c * NP + s * rpt, rpt)])


_edge_pass = functools.partial(
    pl.kernel,
    out_type=jax.ShapeDtypeStruct((NC * NP, WROW), jnp.float32),
    mesh=plsc.VectorSubcoreMesh(core_axis_name="c", subcore_axis_name="s"),
    compiler_params=pltpu.CompilerParams(use_tc_tiling_on_sc=False),
    scratch_types=[
        pltpu.VMEM((C,), jnp.int32),
        pltpu.VMEM((C,), jnp.int32),
        pltpu.VMEM((C, WROW), jnp.float32),
        pltpu.VMEM((C, ERW), jnp.float32),
        pltpu.VMEM((C,), jnp.int32),
        pltpu.VMEM((C,), jnp.int32),
        pltpu.VMEM((C, WROW), jnp.float32),
        pltpu.VMEM((C, ERW), jnp.float32),
        pltpu.VMEM((C, WROW), jnp.float32),
        pltpu.VMEM_SHARED((NP, WROW), jnp.float32),
        pltpu.SemaphoreType.DMA,
        pltpu.SemaphoreType.DMA,
        pltpu.SemaphoreType.DMA,
        pltpu.SemaphoreType.DMA,
    ],
)(_edge_body)


def _gat_out_block(a0_ref, a1_ref, b_ref):
    acc = a0_ref[...] + a1_ref[...]
    ssum = acc[:, HID:HID + H]
    ssum = jnp.where(ssum > 0, ssum, 1.0)
    o = acc[:, :HID].reshape(RB, H, D) / ssum[:, :, None]
    return o.reshape(RB, HID) + b_ref[...]


def _finalize_body(a0_ref, a1_ref, b_ref, g_ref, be_ref, out_ref, st_ref):
    p = pl.program_id(0)
    i = pl.program_id(1)
    o = _gat_out_block(a0_ref, a1_ref, b_ref)

    @pl.when(jnp.logical_and(p == 0, i == 0))
    def _():
        st_ref[...] = jnp.zeros_like(st_ref)

    @pl.when(p == 0)
    def _():
        st_ref[0, :] += jnp.sum(o, axis=0)
        st_ref[1, :] += jnp.sum(o * o, axis=0)

    @pl.when(p == 1)
    def _():
        mu = st_ref[0, :] / N
        var = st_ref[1, :] / N - mu * mu
        on = (o - mu) / jnp.sqrt(var + 1e-3) * g_ref[...] + be_ref[...]
        out_ref[...] = jnp.maximum(on, 0.0)


def _finalize(a0, a1, b, g, be):
    return pl.pallas_call(
        _finalize_body,
        grid=(2, NG),
        in_specs=[
            pl.BlockSpec((RB, WROW), lambda p, i: (i, 0)),
            pl.BlockSpec((RB, WROW), lambda p, i: (i, 0)),
            pl.BlockSpec((HID,), lambda p, i: (0,)),
            pl.BlockSpec((HID,), lambda p, i: (0,)),
            pl.BlockSpec((HID,), lambda p, i: (0,)),
        ],
        out_specs=pl.BlockSpec((RB, HID), lambda p, i: (i, 0)),
        out_shape=jax.ShapeDtypeStruct((N, HID), jnp.float32),
        scratch_shapes=[pltpu.VMEM((8, HID), jnp.float32)],
    )(a0, a1, b, g, be)


def _head_body(a0_ref, a1_ref, b_ref, g_ref, be_ref, h1_ref, lw_ref, lb_ref,
               mask_ref, out_ref, st_ref):
    p = pl.program_id(0)
    i = pl.program_id(1)

    @pl.when(p < 2)
    def _():
        o = _gat_out_block(a0_ref, a1_ref, b_ref)

        @pl.when(jnp.logical_and(p == 0, i == 0))
        def _():
            st_ref[...] = jnp.zeros_like(st_ref)

        @pl.when(p == 0)
        def _():
            st_ref[0, :] += jnp.sum(o, axis=0)
            st_ref[1, :] += jnp.sum(o * o, axis=0)

        @pl.when(p == 1)
        def _():
            mu = st_ref[0, :] / N
            var = st_ref[1, :] / N - mu * mu
            on = (o - mu) / jnp.sqrt(var + 1e-3) * g_ref[...] + be_ref[...]
            h2 = jnp.maximum(on, 0.0) + h1_ref[...]
            st_ref[2, :] += jnp.sum(h2, axis=0)

    @pl.when(p == 2)
    def _():
        pooled = (st_ref[2, :] / N)[None, :]
        m = (mask_ref[...] > 0.5).astype(jnp.float32)
        w = lw_ref[...] * m
        out_ref[...] = (
            jnp.dot(pooled, w.T, preferred_element_type=jnp.float32)
            + lb_ref[...][None, :])


def _head(a0, a1, b, g, be, h1, lw, lb, mask_real):
    return pl.pallas_call(
        _head_body,
        grid=(3, NG),
        in_specs=[
            pl.BlockSpec((RB, WROW), lambda p, i: (i, 0)),
            pl.BlockSpec((RB, WROW), lambda p, i: (i, 0)),
            pl.BlockSpec((HID,), lambda p, i: (0,)),
            pl.BlockSpec((HID,), lambda p, i: (0,)),
            pl.BlockSpec((HID,), lambda p, i: (0,)),
            pl.BlockSpec((RB, HID), lambda p, i: (i, 0)),
            pl.BlockSpec((OUT, HID), lambda p, i: (0, 0)),
            pl.BlockSpec((OUT,), lambda p, i: (0,)),
            pl.BlockSpec((OUT, HID), lambda p, i: (0, 0)),
        ],
        out_specs=pl.BlockSpec((1, OUT), lambda p, i: (0, 0)),
        out_shape=jax.ShapeDtypeStruct((1, OUT), jnp.float32),
        scratch_shapes=[pltpu.VMEM((8, HID), jnp.float32)],
    )(a0, a1, b, g, be, h1, lw, lb, mask_real)


def kernel(h, edge_index, W1, al1, ar1, b1, g1, be1, W2, al2, ar2, b2, g2,
           be2, lw, lb, mask_real):
    src = edge_index[0]
    dst = edge_index[1]
    zeros = jnp.zeros((NP, WROW), dtype=jnp.float32)

    big1, er1 = _prep(h, W1, al1, ar1)
    acc1 = _edge_pass(big1, er1, src, dst, zeros)
    h1 = _finalize(acc1[:N], acc1[NP:NP + N], b1, g1, be1)

    big2, er2 = _prep(h1, W2, al2, ar2)
    acc2 = _edge_pass(big2, er2, src, dst, zeros)
    return _head(acc2[:N], acc2[NP:NP + N], b2, g2, be2, h1, lw, lb,
                 mask_real)


# speedup vs baseline: 1.6615x; 1.2602x over previous
import functools

import jax
import jax.numpy as jnp
from jax import lax
from jax.experimental import pallas as pl
from jax.experimental.pallas import tpu as pltpu
from jax.experimental.pallas import tpu_sc as plsc

N = 10000
E = 320000
D_IN = 128
H = 8
D = 16
HID = H * D
OUT = 64
WROW = HID + 16
ERW = 16

NP = 10240
NC = 2
NS = 16
NW = NC * NS
EPW = E // NW
C = 80
NCHUNK = EPW // C
CSUP = 25
SUPC = CSUP * C
NSUP = NCHUNK // CSUP
NPAIR_L = (CSUP - 1) // 2


def _bcast_lane(v, h):
    idx = jnp.full((16, 1), h, dtype=jnp.int32)
    dn = lax.GatherDimensionNumbers(
        offset_dims=(), collapsed_slice_dims=(0,), start_index_map=(0,))
    return lax.gather(v, idx, dn, (1,),
                      mode=lax.GatherScatterMode.PROMISE_IN_BOUNDS)


RB = 2000
NG = N // RB


def _prep_body(x_ref, w_ref, al_ref, ar_ref, big_ref, er_ref):
    feat = jnp.dot(x_ref[...], w_ref[...], preferred_element_type=jnp.float32)
    f3 = feat.reshape(RB, H, D)
    el = jnp.sum(f3 * al_ref[...][None], axis=-1)
    er = jnp.sum(f3 * ar_ref[...][None], axis=-1)
    z8 = jnp.zeros((RB, 8), dtype=jnp.float32)
    big_ref[...] = jnp.concatenate([feat, el, z8], axis=1)
    er_ref[...] = jnp.concatenate([er, z8], axis=1)


def _prep(x, w, al, ar):
    return pl.pallas_call(
        _prep_body,
        grid=(NG,),
        in_specs=[
            pl.BlockSpec((RB, D_IN), lambda i: (i, 0)),
            pl.BlockSpec((D_IN, HID), lambda i: (0, 0)),
            pl.BlockSpec((H, D), lambda i: (0, 0)),
            pl.BlockSpec((H, D), lambda i: (0, 0)),
        ],
        out_specs=[
            pl.BlockSpec((RB, WROW), lambda i: (i, 0)),
            pl.BlockSpec((RB, ERW), lambda i: (i, 0)),
        ],
        out_shape=[
            jax.ShapeDtypeStruct((N, WROW), jnp.float32),
            jax.ShapeDtypeStruct((N, ERW), jnp.float32),
        ],
    )(x, w, al, ar)


def _edge_body(big_hbm, er_hbm, ei_hbm, zero_hbm, out_hbm,
               rows_a, er_a, rows_b, er_b,
               idx_v, acc_sh, sa1, sa2, sb1, sb2):
    c = lax.axis_index("c")
    s = lax.axis_index("s")
    wid = s * NC + c
    rpt = NP // NS

    pltpu.sync_copy(zero_hbm.at[pl.ds(s * rpt, rpt)],
                    acc_sh.at[pl.ds(s * rpt, rpt)])
    plsc.subcore_barrier()

    def fetch(i, rows_v, er_v, s1, s2):
        pltpu.async_copy(big_hbm.at[idx_v.at[0, pl.ds(i * C, C)]], rows_v, s1)
        pltpu.async_copy(er_hbm.at[idx_v.at[1, pl.ds(i * C, C)]], er_v, s2)

    def wait(i, rows_v, er_v, s1, s2):
        pltpu.make_async_copy(
            big_hbm.at[idx_v.at[0, pl.ds(i * C, C)]], rows_v, s1).wait()
        pltpu.make_async_copy(
            er_hbm.at[idx_v.at[1, pl.ds(i * C, C)]], er_v, s2).wait()

    def compute(i, rows_v, er_v):
        @plsc.parallel_loop(0, C, 1, unroll=4)
        def _edge(e):
            el16 = rows_v[e, pl.ds(HID, 16)]
            er16 = er_v[e, :]
            x = el16 + er16
            ev = jnp.where(x >= 0, x, 0.2 * x)
            ee = jnp.exp(ev)
            for h in range(H):
                bv = _bcast_lane(ee, h)
                rows_v[e, pl.ds(h * D, D)] = rows_v[e, pl.ds(h * D, D)] * bv
            rows_v[e, pl.ds(HID, 16)] = ee
        pltpu.sync_copy(rows_v, acc_sh.at[idx_v.at[1, pl.ds(i * C, C)]],
                        add=True)

    def super_body(su, carry):
        pltpu.sync_copy(ei_hbm.at[:, pl.ds(wid * EPW + su * SUPC, SUPC)],
                        idx_v)
        fetch(0, rows_a, er_a, sa1, sa2)

        def pair(p, carry2):
            fetch(2 * p + 1, rows_b, er_b, sb1, sb2)
            wait(2 * p, rows_a, er_a, sa1, sa2)
            compute(2 * p, rows_a, er_a)
            fetch(2 * p + 2, rows_a, er_a, sa1, sa2)
            wait(2 * p + 1, rows_b, er_b, sb1, sb2)
            compute(2 * p + 1, rows_b, er_b)
            return carry2

        lax.fori_loop(0, NPAIR_L, pair, 0)
        wait(CSUP - 1, rows_a, er_a, sa1, sa2)
        compute(CSUP - 1, rows_a, er_a)
        return carry

    lax.fori_loop(0, NSUP, super_body, 0)

    plsc.subcore_barrier()
    pltpu.sync_copy(acc_sh.at[pl.ds(s * rpt, rpt)],
                    out_hbm.at[pl.ds(c * NP + s * rpt, rpt)])


_edge_pass = functools.partial(
    pl.kernel,
    out_type=jax.ShapeDtypeStruct((NC * NP, WROW), jnp.float32),
    mesh=plsc.VectorSubcoreMesh(core_axis_name="c", subcore_axis_name="s"),
    compiler_params=pltpu.CompilerParams(use_tc_tiling_on_sc=False),
    scratch_types=[
        pltpu.VMEM((C, WROW), jnp.float32),
        pltpu.VMEM((C, ERW), jnp.float32),
        pltpu.VMEM((C, WROW), jnp.float32),
        pltpu.VMEM((C, ERW), jnp.float32),
        pltpu.VMEM((2, SUPC), jnp.int32),
        pltpu.VMEM_SHARED((NP, WROW), jnp.float32),
        pltpu.SemaphoreType.DMA,
        pltpu.SemaphoreType.DMA,
        pltpu.SemaphoreType.DMA,
        pltpu.SemaphoreType.DMA,
    ],
)(_edge_body)


def _gat_out_block(a0_ref, a1_ref, b_ref):
    acc = a0_ref[...] + a1_ref[...]
    ssum = acc[:, HID:HID + H]
    ssum = jnp.where(ssum > 0, ssum, 1.0)
    o = acc[:, :HID].reshape(RB, H, D) / ssum[:, :, None]
    return o.reshape(RB, HID) + b_ref[...]


def _finalize_body(a0_ref, a1_ref, b_ref, g_ref, be_ref, out_ref, st_ref):
    p = pl.program_id(0)
    i = pl.program_id(1)
    o = _gat_out_block(a0_ref, a1_ref, b_ref)

    @pl.when(jnp.logical_and(p == 0, i == 0))
    def _():
        st_ref[...] = jnp.zeros_like(st_ref)

    @pl.when(p == 0)
    def _():
        st_ref[0, :] += jnp.sum(o, axis=0)
        st_ref[1, :] += jnp.sum(o * o, axis=0)

    @pl.when(p == 1)
    def _():
        mu = st_ref[0, :] / N
        var = st_ref[1, :] / N - mu * mu
        on = (o - mu) / jnp.sqrt(var + 1e-3) * g_ref[...] + be_ref[...]
        out_ref[...] = jnp.maximum(on, 0.0)


def _finalize(a0, a1, b, g, be):
    return pl.pallas_call(
        _finalize_body,
        grid=(2, NG),
        in_specs=[
            pl.BlockSpec((RB, WROW), lambda p, i: (i, 0)),
            pl.BlockSpec((RB, WROW), lambda p, i: (i, 0)),
            pl.BlockSpec((HID,), lambda p, i: (0,)),
            pl.BlockSpec((HID,), lambda p, i: (0,)),
            pl.BlockSpec((HID,), lambda p, i: (0,)),
        ],
        out_specs=pl.BlockSpec((RB, HID), lambda p, i: (i, 0)),
        out_shape=jax.ShapeDtypeStruct((N, HID), jnp.float32),
        scratch_shapes=[pltpu.VMEM((8, HID), jnp.float32)],
    )(a0, a1, b, g, be)


def _head_body(a0_ref, a1_ref, b_ref, g_ref, be_ref, h1_ref, lw_ref, lb_ref,
               mask_ref, out_ref, st_ref):
    p = pl.program_id(0)
    i = pl.program_id(1)

    @pl.when(p < 2)
    def _():
        o = _gat_out_block(a0_ref, a1_ref, b_ref)

        @pl.when(jnp.logical_and(p == 0, i == 0))
        def _():
            st_ref[...] = jnp.zeros_like(st_ref)

        @pl.when(p == 0)
        def _():
            st_ref[0, :] += jnp.sum(o, axis=0)
            st_ref[1, :] += jnp.sum(o * o, axis=0)

        @pl.when(p == 1)
        def _():
            mu = st_ref[0, :] / N
            var = st_ref[1, :] / N - mu * mu
            on = (o - mu) / jnp.sqrt(var + 1e-3) * g_ref[...] + be_ref[...]
            h2 = jnp.maximum(on, 0.0) + h1_ref[...]
            st_ref[2, :] += jnp.sum(h2, axis=0)

    @pl.when(p == 2)
    def _():
        pooled = (st_ref[2, :] / N)[None, :]
        m = (mask_ref[...] > 0.5).astype(jnp.float32)
        w = lw_ref[...] * m
        out_ref[...] = (
            jnp.dot(pooled, w.T, preferred_element_type=jnp.float32)
            + lb_ref[...][None, :])


def _head(a0, a1, b, g, be, h1, lw, lb, mask_real):
    return pl.pallas_call(
        _head_body,
        grid=(3, NG),
        in_specs=[
            pl.BlockSpec((RB, WROW), lambda p, i: (i, 0)),
            pl.BlockSpec((RB, WROW), lambda p, i: (i, 0)),
            pl.BlockSpec((HID,), lambda p, i: (0,)),
            pl.BlockSpec((HID,), lambda p, i: (0,)),
            pl.BlockSpec((HID,), lambda p, i: (0,)),
            pl.BlockSpec((RB, HID), lambda p, i: (i, 0)),
            pl.BlockSpec((OUT, HID), lambda p, i: (0, 0)),
            pl.BlockSpec((OUT,), lambda p, i: (0,)),
            pl.BlockSpec((OUT, HID), lambda p, i: (0, 0)),
        ],
        out_specs=pl.BlockSpec((1, OUT), lambda p, i: (0, 0)),
        out_shape=jax.ShapeDtypeStruct((1, OUT), jnp.float32),
        scratch_shapes=[pltpu.VMEM((8, HID), jnp.float32)],
    )(a0, a1, b, g, be, h1, lw, lb, mask_real)


def kernel(h, edge_index, W1, al1, ar1, b1, g1, be1, W2, al2, ar2, b2, g2,
           be2, lw, lb, mask_real):
    zeros = jnp.zeros((NP, WROW), dtype=jnp.float32)

    big1, er1 = _prep(h, W1, al1, ar1)
    acc1 = _edge_pass(big1, er1, edge_index, zeros)
    h1 = _finalize(acc1[:N], acc1[NP:NP + N], b1, g1, be1)

    big2, er2 = _prep(h1, W2, al2, ar2)
    acc2 = _edge_pass(big2, er2, edge_index, zeros)
    return _head(acc2[:N], acc2[NP:NP + N], b2, g2, be2, h1, lw, lb,
                 mask_real)
